# baseline (speedup 1.0000x reference)
import functools

import jax
import jax.numpy as jnp
from jax import lax
from jax.experimental import pallas as pl
from jax.experimental.pallas import tpu as pltpu
from jax.experimental.pallas import tpu_sc as plsc

B, N, D = 8, 576, 256
Q, K = 8, 8192
M = B * N

M_T = 576
K_T = 512
_BIG = jnp.int32(2**30)

_SC_CORES = 2
_SC_SUBCORES = 16
_NW = _SC_CORES * _SC_SUBCORES
_RPW = M // _NW
_GC = 72


def _argmin_body(res_ref, cb_ref, esq_ref, idx_ref, mind_ref):
    r = res_ref[...]
    rsq = jnp.sum(r * r, axis=1)

    def body(k, carry):
        run_min, run_idx = carry
        cb = cb_ref[pl.ds(k * K_T, K_T), :]
        esq = esq_ref[0, pl.ds(k * K_T, K_T)]
        mm = lax.dot_general(
            r, cb, (((1,), (1,)), ((), ())),
            preferred_element_type=jnp.float32,
        )
        d = (rsq[:, None] - 2.0 * mm) + esq[None, :]
        tmin = jnp.min(d, axis=1)
        iota = lax.broadcasted_iota(jnp.int32, (M_T, K_T), 1)
        tidx = jnp.min(jnp.where(d == tmin[:, None], iota, 2**30), axis=1)
        tidx = tidx + k * K_T
        upd = tmin < run_min
        return jnp.where(upd, tmin, run_min), jnp.where(upd, tidx, run_idx)

    init = (jnp.full((M_T,), jnp.inf, jnp.float32),
            jnp.zeros((M_T,), jnp.int32))
    run_min, run_idx = lax.fori_loop(0, K // K_T, body, init)
    idx_ref[0, 0, :] = run_idx
    mind_ref[0, 0, :] = run_min


def _argmin_stage(residual, cb, esq):
    n_mt = M // M_T
    idx3, mind3 = pl.pallas_call(
        _argmin_body,
        grid=(n_mt,),
        in_specs=[
            pl.BlockSpec((M_T, D), lambda i: (i, 0)),
            pl.BlockSpec((K, D), lambda i: (0, 0)),
            pl.BlockSpec((1, K), lambda i: (0, 0)),
        ],
        out_specs=[
            pl.BlockSpec((1, 1, M_T), lambda i: (i, 0, 0)),
            pl.BlockSpec((1, 1, M_T), lambda i: (i, 0, 0)),
        ],
        out_shape=[
            jax.ShapeDtypeStruct((n_mt, 1, M_T), jnp.int32),
            jax.ShapeDtypeStruct((n_mt, 1, M_T), jnp.float32),
        ],
    )(residual, cb, esq)
    return idx3.reshape(M), mind3.reshape(M)


def _make_sc_gather(last):
    mesh = plsc.VectorSubcoreMesh(
        core_axis_name="c", subcore_axis_name="s", num_cores=_SC_CORES)
    n_out = 2 if last else 1
    out_type = [jax.ShapeDtypeStruct((M, D), jnp.float32)] * n_out
    scratch = [
        pltpu.VMEM((_RPW,), jnp.int32),
        pltpu.VMEM((_RPW, D), jnp.float32),
        pltpu.VMEM((_RPW, D), jnp.float32),
        pltpu.SemaphoreType.DMA,
    ]
    if last:
        scratch.append(pltpu.VMEM((_RPW, D), jnp.float32))

    def _sub_loop(dst_v, a_v, b_v):
        def row(i, _):
            def col(j, _):
                s = pl.ds(j * 16, 16)
                dst_v[i, s] = a_v[i, s] - b_v[i, s]
                return 0
            return lax.fori_loop(0, D // 16, col, 0)
        lax.fori_loop(0, _RPW, row, 0)

    def body(table_hbm, idx_hbm, res_hbm, *rest):
        if last:
            x_hbm, res_out, q_out, idx_v, rows_v, res_v, sem, x_v = rest
        else:
            res_out, idx_v, rows_v, res_v, sem = rest
        wid = lax.axis_index("s") * _SC_CORES + lax.axis_index("c")
        base = wid * _RPW
        pltpu.sync_copy(idx_hbm.at[pl.ds(base, _RPW)], idx_v)
        for c in range(_RPW // _GC):
            s = pl.ds(c * _GC, _GC)
            pltpu.async_copy(table_hbm.at[idx_v.at[s]], rows_v.at[s], sem).wait()
        pltpu.sync_copy(res_hbm.at[pl.ds(base, _RPW)], res_v)
        _sub_loop(res_v, res_v, rows_v)
        pltpu.sync_copy(res_v, res_out.at[pl.ds(base, _RPW)])
        if last:
            pltpu.sync_copy(x_hbm.at[pl.ds(base, _RPW)], x_v)
            _sub_loop(rows_v, x_v, res_v)
            pltpu.sync_copy(rows_v, q_out.at[pl.ds(base, _RPW)])

    return pl.kernel(body, mesh=mesh, out_type=out_type, scratch_types=scratch)


_sc_gather_mid = _make_sc_gather(last=False)
_sc_gather_last = _make_sc_gather(last=True)


def kernel(x, codebooks):
    xf = x.reshape(M, D)
    esqs = jnp.sum(codebooks ** 2, axis=-1)
    residual = xf
    idx_list, mind_list = [], []
    qout = None
    for q in range(Q):
        idx, mind = _argmin_stage(residual, codebooks[q],
                                  esqs[q].reshape(1, K))
        idx_list.append(idx)
        mind_list.append(mind)
        if q < Q - 1:
            (residual,) = _sc_gather_mid(codebooks[q], idx, residual)
        else:
            residual, qout = _sc_gather_last(codebooks[q], idx, residual, xf)
    all_idx = jnp.stack(idx_list, axis=-1).reshape(B, N, Q)
    minds = jnp.stack(mind_list, axis=-1).reshape(B, N, Q).astype(jnp.float32)
    losses = jnp.sum(minds, axis=1) / (N * D)
    return qout.reshape(B, N, D), all_idx, losses
